# baseline (speedup 1.0000x reference)
import jax
import jax.numpy as jnp
from jax.experimental import pallas as pl
from jax.experimental.pallas import tpu as pltpu

_PAR = pltpu.CompilerParams(dimension_semantics=("parallel",))


def _support_body(x_ref, w_ref, out_ref):
    out_ref[...] = jnp.dot(
        x_ref[...].astype(jnp.bfloat16), w_ref[...],
        preferred_element_type=jnp.float32).astype(jnp.bfloat16)


def _pass1_body(adj_ref, s_ref, b_ref, w_ref, s_next_ref, adj16_ref):
    a16 = adj_ref[...].astype(jnp.bfloat16)
    adj16_ref[...] = a16
    acc = jnp.dot(a16, s_ref[...], preferred_element_type=jnp.float32)
    h = jnp.maximum(acc + b_ref[...], 0.0).astype(jnp.bfloat16)
    s_next_ref[...] = jnp.dot(
        h, w_ref[...], preferred_element_type=jnp.float32).astype(jnp.bfloat16)


def _mid_body(adj16_ref, s_ref, b_ref, w_ref, s_next_ref):
    acc = jnp.dot(adj16_ref[...], s_ref[...], preferred_element_type=jnp.float32)
    h = jnp.maximum(acc + b_ref[...], 0.0).astype(jnp.bfloat16)
    s_next_ref[...] = jnp.dot(
        h, w_ref[...], preferred_element_type=jnp.float32).astype(jnp.bfloat16)


def _final_body(adj16_ref, s_ref, b_ref, out_ref):
    acc = jnp.dot(adj16_ref[...], s_ref[...], preferred_element_type=jnp.float32)
    out_ref[...] = acc + b_ref[...]


def _rows_block(n, target):
    for bi in range(min(target, n), 7, -1):
        if n % bi == 0 and bi % 8 == 0:
            return bi
    return n


def kernel(x, adj, W1, b1, W2, b2, W3, b3):
    n, in_c = x.shape
    h1 = W1.shape[1]
    h2 = W2.shape[1]
    out_c = W3.shape[1]
    f32 = jnp.float32
    bf16 = jnp.bfloat16

    bs = _rows_block(n, 2000)
    bi1 = _rows_block(n, 200)
    bi = _rows_block(n, 400)

    w1_16 = W1.astype(bf16)
    w2_16 = W2.astype(bf16)
    w3_16 = W3.astype(bf16)
    b1r = b1.reshape(1, h1)
    b2r = b2.reshape(1, h2)
    b3r = b3.reshape(1, out_c)

    s1 = pl.pallas_call(
        _support_body,
        grid=(n // bs,),
        in_specs=[pl.BlockSpec((bs, in_c), lambda i: (i, 0)),
                  pl.BlockSpec((in_c, h1), lambda i: (0, 0))],
        out_specs=pl.BlockSpec((bs, h1), lambda i: (i, 0)),
        out_shape=jax.ShapeDtypeStruct((n, h1), bf16),
        compiler_params=_PAR,
    )(x, w1_16)

    s2, adj16 = pl.pallas_call(
        _pass1_body,
        grid=(n // bi1,),
        in_specs=[pl.BlockSpec((bi1, n), lambda i: (i, 0)),
                  pl.BlockSpec((n, h1), lambda i: (0, 0)),
                  pl.BlockSpec((1, h1), lambda i: (0, 0)),
                  pl.BlockSpec((h1, h2), lambda i: (0, 0))],
        out_specs=[pl.BlockSpec((bi1, h2), lambda i: (i, 0)),
                   pl.BlockSpec((bi1, n), lambda i: (i, 0))],
        out_shape=[jax.ShapeDtypeStruct((n, h2), bf16),
                   jax.ShapeDtypeStruct((n, n), bf16)],
        compiler_params=_PAR,
    )(adj, s1, b1r, w2_16)

    s3 = pl.pallas_call(
        _mid_body,
        grid=(n // bi,),
        in_specs=[pl.BlockSpec((bi, n), lambda i: (i, 0)),
                  pl.BlockSpec((n, h2), lambda i: (0, 0)),
                  pl.BlockSpec((1, h2), lambda i: (0, 0)),
                  pl.BlockSpec((h2, out_c), lambda i: (0, 0))],
        out_specs=pl.BlockSpec((bi, out_c), lambda i: (i, 0)),
        out_shape=jax.ShapeDtypeStruct((n, out_c), bf16),
        compiler_params=_PAR,
    )(adj16, s2, b2r, w3_16)

    out = pl.pallas_call(
        _final_body,
        grid=(n // bi,),
        in_specs=[pl.BlockSpec((bi, n), lambda i: (i, 0)),
                  pl.BlockSpec((n, out_c), lambda i: (0, 0)),
                  pl.BlockSpec((1, out_c), lambda i: (0, 0))],
        out_specs=pl.BlockSpec((bi, out_c), lambda i: (i, 0)),
        out_shape=jax.ShapeDtypeStruct((n, out_c), f32),
        compiler_params=_PAR,
    )(adj16, s3, b3r)

    return out


# speedup vs baseline: 1.0389x; 1.0389x over previous
import jax
import jax.numpy as jnp
from jax.experimental import pallas as pl
from jax.experimental.pallas import tpu as pltpu

_PAR = pltpu.CompilerParams(dimension_semantics=("parallel",))


def _support_body(x_ref, w_ref, out_ref):
    out_ref[...] = jnp.dot(
        x_ref[...].astype(jnp.bfloat16), w_ref[...],
        preferred_element_type=jnp.float32).astype(jnp.bfloat16)


def _pass1_body(adj_ref, s_ref, b_ref, w_ref, s_next_ref, adj16_ref):
    a16 = adj_ref[...].astype(jnp.bfloat16)
    adj16_ref[...] = a16
    acc = jnp.dot(a16, s_ref[...], preferred_element_type=jnp.float32)
    h = jnp.maximum(acc + b_ref[...], 0.0).astype(jnp.bfloat16)
    s_next_ref[...] = jnp.dot(
        h, w_ref[...], preferred_element_type=jnp.float32).astype(jnp.bfloat16)


def _mid_body(adj16_ref, s_ref, b_ref, w_ref, s_next_ref):
    acc = jnp.dot(adj16_ref[...], s_ref[...], preferred_element_type=jnp.float32)
    h = jnp.maximum(acc + b_ref[...], 0.0).astype(jnp.bfloat16)
    s_next_ref[...] = jnp.dot(
        h, w_ref[...], preferred_element_type=jnp.float32).astype(jnp.bfloat16)


def _final_body(adj16_ref, s_ref, b_ref, out_ref):
    acc = jnp.dot(adj16_ref[...], s_ref[...], preferred_element_type=jnp.float32)
    out_ref[...] = acc + b_ref[...]


def _rows_block(n, target):
    for bi in range(min(target, n), 7, -1):
        if n % bi == 0 and bi % 8 == 0:
            return bi
    return n


def kernel(x, adj, W1, b1, W2, b2, W3, b3):
    n, in_c = x.shape
    h1 = W1.shape[1]
    h2 = W2.shape[1]
    out_c = W3.shape[1]
    f32 = jnp.float32
    bf16 = jnp.bfloat16

    bs = _rows_block(n, 2000)
    bi1 = _rows_block(n, 200)
    bi = _rows_block(n, 1000)

    w1_16 = W1.astype(bf16)
    w2_16 = W2.astype(bf16)
    w3_16 = W3.astype(bf16)
    b1r = b1.reshape(1, h1)
    b2r = b2.reshape(1, h2)
    b3r = b3.reshape(1, out_c)

    s1 = pl.pallas_call(
        _support_body,
        grid=(n // bs,),
        in_specs=[pl.BlockSpec((bs, in_c), lambda i: (i, 0)),
                  pl.BlockSpec((in_c, h1), lambda i: (0, 0))],
        out_specs=pl.BlockSpec((bs, h1), lambda i: (i, 0)),
        out_shape=jax.ShapeDtypeStruct((n, h1), bf16),
        compiler_params=_PAR,
    )(x, w1_16)

    s2, adj16 = pl.pallas_call(
        _pass1_body,
        grid=(n // bi1,),
        in_specs=[pl.BlockSpec((bi1, n), lambda i: (i, 0)),
                  pl.BlockSpec((n, h1), lambda i: (0, 0)),
                  pl.BlockSpec((1, h1), lambda i: (0, 0)),
                  pl.BlockSpec((h1, h2), lambda i: (0, 0))],
        out_specs=[pl.BlockSpec((bi1, h2), lambda i: (i, 0)),
                   pl.BlockSpec((bi1, n), lambda i: (i, 0))],
        out_shape=[jax.ShapeDtypeStruct((n, h2), bf16),
                   jax.ShapeDtypeStruct((n, n), bf16)],
        compiler_params=_PAR,
    )(adj, s1, b1r, w2_16)

    s3 = pl.pallas_call(
        _mid_body,
        grid=(n // bi,),
        in_specs=[pl.BlockSpec((bi, n), lambda i: (i, 0)),
                  pl.BlockSpec((n, h2), lambda i: (0, 0)),
                  pl.BlockSpec((1, h2), lambda i: (0, 0)),
                  pl.BlockSpec((h2, out_c), lambda i: (0, 0))],
        out_specs=pl.BlockSpec((bi, out_c), lambda i: (i, 0)),
        out_shape=jax.ShapeDtypeStruct((n, out_c), bf16),
        compiler_params=_PAR,
    )(adj16, s2, b2r, w3_16)

    out = pl.pallas_call(
        _final_body,
        grid=(n // bi,),
        in_specs=[pl.BlockSpec((bi, n), lambda i: (i, 0)),
                  pl.BlockSpec((n, out_c), lambda i: (0, 0)),
                  pl.BlockSpec((1, out_c), lambda i: (0, 0))],
        out_specs=pl.BlockSpec((bi, out_c), lambda i: (i, 0)),
        out_shape=jax.ShapeDtypeStruct((n, out_c), f32),
        compiler_params=_PAR,
    )(adj16, s3, b3r)

    return out


# speedup vs baseline: 1.0422x; 1.0032x over previous
import jax
import jax.numpy as jnp
from jax.experimental import pallas as pl
from jax.experimental.pallas import tpu as pltpu

_PAR = pltpu.CompilerParams(dimension_semantics=("parallel",))


def _support_body(x_ref, w_ref, out_ref):
    out_ref[...] = jnp.dot(
        x_ref[...].astype(jnp.bfloat16), w_ref[...],
        preferred_element_type=jnp.float32).astype(jnp.bfloat16)


def _pass1_body(adj_ref, s_ref, b_ref, w_ref, s_next_ref, adj16_ref):
    a16 = adj_ref[...].astype(jnp.bfloat16)
    adj16_ref[...] = a16
    acc = jnp.dot(a16, s_ref[...], preferred_element_type=jnp.float32)
    h = jnp.maximum(acc + b_ref[...], 0.0).astype(jnp.bfloat16)
    s_next_ref[...] = jnp.dot(
        h, w_ref[...], preferred_element_type=jnp.float32).astype(jnp.bfloat16)


def _mid_body(adj16_ref, s_ref, b_ref, w_ref, s_next_ref):
    acc = jnp.dot(adj16_ref[...], s_ref[...], preferred_element_type=jnp.float32)
    h = jnp.maximum(acc + b_ref[...], 0.0).astype(jnp.bfloat16)
    s_next_ref[...] = jnp.dot(
        h, w_ref[...], preferred_element_type=jnp.float32).astype(jnp.bfloat16)


def _final_body(adj16_ref, s_ref, b_ref, out_ref):
    acc = jnp.dot(adj16_ref[...], s_ref[...], preferred_element_type=jnp.float32)
    out_ref[...] = acc + b_ref[...]


def _rows_block(n, target):
    for bi in range(min(target, n), 7, -1):
        if n % bi == 0 and bi % 8 == 0:
            return bi
    return n


def kernel(x, adj, W1, b1, W2, b2, W3, b3):
    n, in_c = x.shape
    h1 = W1.shape[1]
    h2 = W2.shape[1]
    out_c = W3.shape[1]
    f32 = jnp.float32
    bf16 = jnp.bfloat16

    bs = _rows_block(n, 2000)
    bi1 = _rows_block(n, 400)
    bi = _rows_block(n, 1000)

    w1_16 = W1.astype(bf16)
    w2_16 = W2.astype(bf16)
    w3_16 = W3.astype(bf16)
    b1r = b1.reshape(1, h1)
    b2r = b2.reshape(1, h2)
    b3r = b3.reshape(1, out_c)

    s1 = pl.pallas_call(
        _support_body,
        grid=(n // bs,),
        in_specs=[pl.BlockSpec((bs, in_c), lambda i: (i, 0)),
                  pl.BlockSpec((in_c, h1), lambda i: (0, 0))],
        out_specs=pl.BlockSpec((bs, h1), lambda i: (i, 0)),
        out_shape=jax.ShapeDtypeStruct((n, h1), bf16),
        compiler_params=_PAR,
    )(x, w1_16)

    s2, adj16 = pl.pallas_call(
        _pass1_body,
        grid=(n // bi1,),
        in_specs=[pl.BlockSpec((bi1, n), lambda i: (i, 0)),
                  pl.BlockSpec((n, h1), lambda i: (0, 0)),
                  pl.BlockSpec((1, h1), lambda i: (0, 0)),
                  pl.BlockSpec((h1, h2), lambda i: (0, 0))],
        out_specs=[pl.BlockSpec((bi1, h2), lambda i: (i, 0)),
                   pl.BlockSpec((bi1, n), lambda i: (i, 0))],
        out_shape=[jax.ShapeDtypeStruct((n, h2), bf16),
                   jax.ShapeDtypeStruct((n, n), bf16)],
        compiler_params=_PAR,
    )(adj, s1, b1r, w2_16)

    s3 = pl.pallas_call(
        _mid_body,
        grid=(n // bi,),
        in_specs=[pl.BlockSpec((bi, n), lambda i: (i, 0)),
                  pl.BlockSpec((n, h2), lambda i: (0, 0)),
                  pl.BlockSpec((1, h2), lambda i: (0, 0)),
                  pl.BlockSpec((h2, out_c), lambda i: (0, 0))],
        out_specs=pl.BlockSpec((bi, out_c), lambda i: (i, 0)),
        out_shape=jax.ShapeDtypeStruct((n, out_c), bf16),
        compiler_params=_PAR,
    )(adj16, s2, b2r, w3_16)

    out = pl.pallas_call(
        _final_body,
        grid=(n // bi,),
        in_specs=[pl.BlockSpec((bi, n), lambda i: (i, 0)),
                  pl.BlockSpec((n, out_c), lambda i: (0, 0)),
                  pl.BlockSpec((1, out_c), lambda i: (0, 0))],
        out_specs=pl.BlockSpec((bi, out_c), lambda i: (i, 0)),
        out_shape=jax.ShapeDtypeStruct((n, out_c), f32),
        compiler_params=_PAR,
    )(adj16, s3, b3r)

    return out
